# baseline (speedup 1.0000x reference)
import functools

import jax
import jax.numpy as jnp
from jax import lax
from jax.experimental import pallas as pl
from jax.experimental.pallas import tpu as pltpu


def _key_to_f32(k):
    m = k ^ ((k >> 31) & jnp.int32(0x7FFFFFFF))
    return lax.bitcast_convert_type(m, jnp.float32)


def _mid(lo, hi):
    return (lo >> 1) + (hi >> 1) + (lo & hi & jnp.int32(1))


def _thresh_body(x_ref, a_ref, t_ref, th_ref, tm_ref, *, r_st, r_en, n_iter,
                 n_sub, sub, c, w):

    def count_le2(ta, tb):
        ta3 = ta.reshape(1, c, 1)
        tb3 = tb.reshape(1, c, 1)

        def cbody(j, accs):
            acc_a, acc_b = accs
            xs = x_ref[0, pl.ds(j * sub, sub), :, :]
            acc_a = acc_a + jnp.sum((xs <= ta3).astype(jnp.float32), axis=0)
            acc_b = acc_b + jnp.sum((xs <= tb3).astype(jnp.float32), axis=0)
            return acc_a, acc_b

        z = jnp.zeros((c, w), jnp.float32)
        acc_a, acc_b = lax.fori_loop(0, n_sub, cbody, (z, z))
        return (jnp.sum(acc_a, axis=1, keepdims=True),
                jnp.sum(acc_b, axis=1, keepdims=True))

    imin = jnp.full((c, 1), jnp.iinfo(jnp.int32).min, jnp.int32)
    imax = jnp.full((c, 1), jnp.iinfo(jnp.int32).max, jnp.int32)
    tgt1 = jnp.float32(r_st + 1)
    tgt2 = jnp.float32(r_en + 1)

    def step(_, state):
        lo1, hi1, lo2, hi2 = state
        m1 = _mid(lo1, hi1)
        m2 = _mid(lo2, hi2)
        c1, c2 = count_le2(_key_to_f32(m1), _key_to_f32(m2))
        p1 = c1 >= tgt1
        p2 = c2 >= tgt2
        lo1 = jnp.where(p1, lo1, m1 + 1)
        hi1 = jnp.where(p1, m1, hi1)
        lo2 = jnp.where(p2, lo2, m2 + 1)
        hi2 = jnp.where(p2, m2, hi2)
        return lo1, hi1, lo2, hi2

    lo1, _, lo2, _ = lax.fori_loop(0, n_iter, step, (imin, imax, imin, imax))
    st = _key_to_f32(lo1)
    en = _key_to_f32(lo2)

    th0 = st + (en - st) * a_ref[0]
    val0 = (th0 > 1e-14).astype(jnp.float32)
    th = th0 * val0
    val_st = th + (1.0 - val0)
    tau_m = t_ref[0] / val_st
    th_ref[0] = th.reshape(1, c, 1)
    tm_ref[0] = tau_m.reshape(1, c, 1)


def _prox_body(x_ref, th_ref, tm_ref, o_ref):
    xb = x_ref[0]
    th = th_ref[0]
    tau_m = tm_ref[0]
    o_ref[0] = jnp.maximum(xb, 0.0) / (
        1.0 + jnp.exp(-tau_m * (jnp.abs(xb) - th)))


def kernel(x, alpha, tau):
    B, H, W, C = x.shape
    L = H * W
    r_st = L - 1 - int(0.99 * L)
    r_en = L - 1 - int(0.01 * L)

    xt = jnp.transpose(x, (0, 1, 3, 2))

    n_sub = 1
    for cand in (8, 7, 4, 2):
        if H % cand == 0 and H // cand >= 8:
            n_sub = cand
            break
    sub = H // n_sub

    tbody = functools.partial(_thresh_body, r_st=r_st, r_en=r_en, n_iter=24,
                              n_sub=n_sub, sub=sub, c=C, w=W)
    th, tm = pl.pallas_call(
        tbody,
        grid=(B,),
        in_specs=[
            pl.BlockSpec((1, H, C, W), lambda b: (b, 0, 0, 0)),
            pl.BlockSpec(memory_space=pltpu.SMEM),
            pl.BlockSpec(memory_space=pltpu.SMEM),
        ],
        out_specs=[
            pl.BlockSpec((1, 1, C, 1), lambda b: (b, 0, 0, 0)),
            pl.BlockSpec((1, 1, C, 1), lambda b: (b, 0, 0, 0)),
        ],
        out_shape=[
            jax.ShapeDtypeStruct((B, 1, C, 1), jnp.float32),
            jax.ShapeDtypeStruct((B, 1, C, 1), jnp.float32),
        ],
    )(xt, alpha, tau)

    y = pl.pallas_call(
        _prox_body,
        grid=(B, n_sub),
        in_specs=[
            pl.BlockSpec((1, sub, C, W), lambda b, j: (b, j, 0, 0)),
            pl.BlockSpec((1, 1, C, 1), lambda b, j: (b, 0, 0, 0)),
            pl.BlockSpec((1, 1, C, 1), lambda b, j: (b, 0, 0, 0)),
        ],
        out_specs=pl.BlockSpec((1, sub, C, W), lambda b, j: (b, j, 0, 0)),
        out_shape=jax.ShapeDtypeStruct((B, H, C, W), jnp.float32),
    )(xt, th, tm)
    return jnp.transpose(y, (0, 1, 3, 2))


# speedup vs baseline: 1.1993x; 1.1993x over previous
import functools

import jax
import jax.numpy as jnp
from jax import lax
from jax.experimental import pallas as pl
from jax.experimental.pallas import tpu as pltpu


def _key_to_f32(k):
    m = k ^ ((k >> 31) & jnp.int32(0x7FFFFFFF))
    return lax.bitcast_convert_type(m, jnp.float32)


def _mid(lo, hi):
    return (lo >> 1) + (hi >> 1) + (lo & hi & jnp.int32(1))


def _thresh_body(x_ref, a_ref, t_ref, th_ref, tm_ref, *, r_st, r_en, n_iter,
                 n_sub, sub, c, w):

    def count_le2(ta, tb):
        ta3 = ta.reshape(1, c, 1)
        tb3 = tb.reshape(1, c, 1)

        def cbody(j, accs):
            acc_a, acc_b = accs
            xs = x_ref[0, pl.ds(j * sub, sub), :, :]
            acc_a = acc_a + jnp.sum((xs <= ta3).astype(jnp.float32), axis=0)
            acc_b = acc_b + jnp.sum((xs <= tb3).astype(jnp.float32), axis=0)
            return acc_a, acc_b

        z = jnp.zeros((c, w), jnp.float32)
        acc_a, acc_b = lax.fori_loop(0, n_sub, cbody, (z, z))
        return (jnp.sum(acc_a, axis=1, keepdims=True),
                jnp.sum(acc_b, axis=1, keepdims=True))

    imin = jnp.full((c, 1), jnp.iinfo(jnp.int32).min, jnp.int32)
    imax = jnp.full((c, 1), jnp.iinfo(jnp.int32).max, jnp.int32)
    tgt1 = jnp.float32(r_st + 1)
    tgt2 = jnp.float32(r_en + 1)

    def step(_, state):
        lo1, hi1, lo2, hi2 = state
        m1 = _mid(lo1, hi1)
        m2 = _mid(lo2, hi2)
        c1, c2 = count_le2(_key_to_f32(m1), _key_to_f32(m2))
        p1 = c1 >= tgt1
        p2 = c2 >= tgt2
        lo1 = jnp.where(p1, lo1, m1 + 1)
        hi1 = jnp.where(p1, m1, hi1)
        lo2 = jnp.where(p2, lo2, m2 + 1)
        hi2 = jnp.where(p2, m2, hi2)
        return lo1, hi1, lo2, hi2

    lo1, _, lo2, _ = lax.fori_loop(0, n_iter, step, (imin, imax, imin, imax))
    st = _key_to_f32(lo1)
    en = _key_to_f32(lo2)

    th0 = st + (en - st) * a_ref[0]
    val0 = (th0 > 1e-14).astype(jnp.float32)
    th = th0 * val0
    val_st = th + (1.0 - val0)
    tau_m = t_ref[0] / val_st
    th_ref[0] = th.reshape(1, c, 1)
    tm_ref[0] = tau_m.reshape(1, c, 1)


def _prox_body(x_ref, th_ref, tm_ref, o_ref):
    xb = x_ref[0]
    th = th_ref[0]
    tau_m = tm_ref[0]
    o_ref[0] = jnp.maximum(xb, 0.0) / (
        1.0 + jnp.exp(-tau_m * (jnp.abs(xb) - th)))


def kernel(x, alpha, tau):
    B, H, W, C = x.shape
    L = H * W
    r_st = L - 1 - int(0.99 * L)
    r_en = L - 1 - int(0.01 * L)

    xt = jnp.transpose(x, (0, 1, 3, 2))

    n_sub = 1
    for cand in (4, 8, 7, 2):
        if H % cand == 0 and H // cand >= 8:
            n_sub = cand
            break
    sub = H // n_sub

    tbody = functools.partial(_thresh_body, r_st=r_st, r_en=r_en, n_iter=20,
                              n_sub=n_sub, sub=sub, c=C, w=W)
    th, tm = pl.pallas_call(
        tbody,
        grid=(B,),
        in_specs=[
            pl.BlockSpec((1, H, C, W), lambda b: (b, 0, 0, 0)),
            pl.BlockSpec(memory_space=pltpu.SMEM),
            pl.BlockSpec(memory_space=pltpu.SMEM),
        ],
        out_specs=[
            pl.BlockSpec((1, 1, C, 1), lambda b: (b, 0, 0, 0)),
            pl.BlockSpec((1, 1, C, 1), lambda b: (b, 0, 0, 0)),
        ],
        out_shape=[
            jax.ShapeDtypeStruct((B, 1, C, 1), jnp.float32),
            jax.ShapeDtypeStruct((B, 1, C, 1), jnp.float32),
        ],
    )(xt, alpha, tau)

    y = pl.pallas_call(
        _prox_body,
        grid=(B, n_sub),
        in_specs=[
            pl.BlockSpec((1, sub, C, W), lambda b, j: (b, j, 0, 0)),
            pl.BlockSpec((1, 1, C, 1), lambda b, j: (b, 0, 0, 0)),
            pl.BlockSpec((1, 1, C, 1), lambda b, j: (b, 0, 0, 0)),
        ],
        out_specs=pl.BlockSpec((1, sub, C, W), lambda b, j: (b, j, 0, 0)),
        out_shape=jax.ShapeDtypeStruct((B, H, C, W), jnp.float32),
    )(xt, th, tm)
    return jnp.transpose(y, (0, 1, 3, 2))


# speedup vs baseline: 1.2069x; 1.0063x over previous
import functools

import jax
import jax.numpy as jnp
from jax import lax
from jax.experimental import pallas as pl
from jax.experimental.pallas import tpu as pltpu


def _key_to_f32(k):
    m = k ^ ((k >> 31) & jnp.int32(0x7FFFFFFF))
    return lax.bitcast_convert_type(m, jnp.float32)


def _mid(lo, hi):
    return (lo >> 1) + (hi >> 1) + (lo & hi & jnp.int32(1))


def _thresh_body(x_ref, a_ref, t_ref, th_ref, tm_ref, *, r_st, r_en, n_iter,
                 n_sub, sub, c, w):

    def count_le2(ta, tb):
        ta3 = ta.reshape(1, c, 1)
        tb3 = tb.reshape(1, c, 1)

        def cbody(j, accs):
            acc_a, acc_b = accs
            xs = x_ref[0, pl.ds(j * sub, sub), :, :]
            acc_a = acc_a + jnp.sum((xs <= ta3).astype(jnp.float32), axis=0)
            acc_b = acc_b + jnp.sum((xs <= tb3).astype(jnp.float32), axis=0)
            return acc_a, acc_b

        z = jnp.zeros((c, w), jnp.float32)
        acc_a, acc_b = lax.fori_loop(0, n_sub, cbody, (z, z))
        return (jnp.sum(acc_a, axis=1, keepdims=True),
                jnp.sum(acc_b, axis=1, keepdims=True))

    imin = jnp.full((c, 1), jnp.iinfo(jnp.int32).min, jnp.int32)
    imax = jnp.full((c, 1), jnp.iinfo(jnp.int32).max, jnp.int32)
    tgt1 = jnp.float32(r_st + 1)
    tgt2 = jnp.float32(r_en + 1)

    def step(_, state):
        lo1, hi1, lo2, hi2 = state
        m1 = _mid(lo1, hi1)
        m2 = _mid(lo2, hi2)
        c1, c2 = count_le2(_key_to_f32(m1), _key_to_f32(m2))
        p1 = c1 >= tgt1
        p2 = c2 >= tgt2
        lo1 = jnp.where(p1, lo1, m1 + 1)
        hi1 = jnp.where(p1, m1, hi1)
        lo2 = jnp.where(p2, lo2, m2 + 1)
        hi2 = jnp.where(p2, m2, hi2)
        return lo1, hi1, lo2, hi2

    lo1, _, lo2, _ = lax.fori_loop(0, n_iter, step, (imin, imax, imin, imax))
    st = _key_to_f32(lo1)
    en = _key_to_f32(lo2)

    th0 = st + (en - st) * a_ref[0]
    val0 = (th0 > 1e-14).astype(jnp.float32)
    th = th0 * val0
    val_st = th + (1.0 - val0)
    tau_m = t_ref[0] / val_st
    th_ref[0] = th.reshape(1, c, 1)
    tm_ref[0] = tau_m.reshape(1, c, 1)


def _prox_body(x_ref, th_ref, tm_ref, o_ref):
    xb = x_ref[0]
    th = th_ref[0]
    tau_m = tm_ref[0]
    o_ref[0] = jnp.maximum(xb, 0.0) / (
        1.0 + jnp.exp(-tau_m * (jnp.abs(xb) - th)))


def kernel(x, alpha, tau):
    B, H, W, C = x.shape
    L = H * W
    r_st = L - 1 - int(0.99 * L)
    r_en = L - 1 - int(0.01 * L)

    xt = jnp.transpose(x, (0, 1, 3, 2))

    n_sub = 1
    for cand in (2, 4, 8, 7):
        if H % cand == 0 and H // cand >= 8:
            n_sub = cand
            break
    sub = H // n_sub

    tbody = functools.partial(_thresh_body, r_st=r_st, r_en=r_en, n_iter=20,
                              n_sub=n_sub, sub=sub, c=C, w=W)
    th, tm = pl.pallas_call(
        tbody,
        grid=(B,),
        in_specs=[
            pl.BlockSpec((1, H, C, W), lambda b: (b, 0, 0, 0)),
            pl.BlockSpec(memory_space=pltpu.SMEM),
            pl.BlockSpec(memory_space=pltpu.SMEM),
        ],
        out_specs=[
            pl.BlockSpec((1, 1, C, 1), lambda b: (b, 0, 0, 0)),
            pl.BlockSpec((1, 1, C, 1), lambda b: (b, 0, 0, 0)),
        ],
        out_shape=[
            jax.ShapeDtypeStruct((B, 1, C, 1), jnp.float32),
            jax.ShapeDtypeStruct((B, 1, C, 1), jnp.float32),
        ],
    )(xt, alpha, tau)

    y = pl.pallas_call(
        _prox_body,
        grid=(B, n_sub),
        in_specs=[
            pl.BlockSpec((1, sub, C, W), lambda b, j: (b, j, 0, 0)),
            pl.BlockSpec((1, 1, C, 1), lambda b, j: (b, 0, 0, 0)),
            pl.BlockSpec((1, 1, C, 1), lambda b, j: (b, 0, 0, 0)),
        ],
        out_specs=pl.BlockSpec((1, sub, C, W), lambda b, j: (b, j, 0, 0)),
        out_shape=jax.ShapeDtypeStruct((B, H, C, W), jnp.float32),
    )(xt, th, tm)
    return jnp.transpose(y, (0, 1, 3, 2))
